# Initial kernel scaffold; baseline (speedup 1.0000x reference)
import functools

import jax
import jax.numpy as jnp
from jax import lax
from jax.experimental import pallas as pl
from jax.experimental.pallas import tpu as pltpu
from jax.experimental.pallas import tpu_sc as plsc

N = 10000
E = 320000
T = 8
D = 128
H = 128

NC = 2
NS = 16
NW = NC * NS
KE = 128
CHUNKS = -(-E // (NW * KE))
EPAD = CHUNKS * NW * KE
PER_TILE = EPAD // NW
NROWS = N // NS

_mesh = plsc.VectorSubcoreMesh(
    core_axis_name="c", subcore_axis_name="s", num_cores=NC, num_subcores=NS)


def _sc_edge_scatter_body(hs_hbm, src_hbm, dst_hbm, ew16_hbm, zer_hbm,
                          out_hbm, src_v, dst_v, ew_v, rows_v, acc_sh, sem):
    c = lax.axis_index("c")
    s = lax.axis_index("s")
    w = c * NS + s
    row0 = s * NROWS
    pltpu.sync_copy(zer_hbm.at[pl.ds(row0, NROWS)], acc_sh.at[pl.ds(row0, NROWS)])
    plsc.subcore_barrier()

    def chunk(g, carry):
        base = w * PER_TILE + g * KE
        pltpu.sync_copy(src_hbm.at[pl.ds(base, KE)], src_v)
        pltpu.sync_copy(dst_hbm.at[pl.ds(base, KE)], dst_v)
        pltpu.sync_copy(ew16_hbm.at[pl.ds(base, KE)], ew_v)
        pltpu.async_copy(hs_hbm.at[src_v], rows_v, sem).wait()

        def scale(jg, carry2):
            for u in range(8):
                j = jg * 8 + u
                wv = ew_v[j]
                for i in range(H // 16):
                    sl = pl.ds(i * 16, 16)
                    rows_v[j, sl] = rows_v[j, sl] * wv
            return carry2

        lax.fori_loop(0, KE // 8, scale, 0)
        pltpu.sync_copy(rows_v, acc_sh.at[dst_v], add=True)
        return carry

    lax.fori_loop(0, CHUNKS, chunk, 0)
    plsc.subcore_barrier()
    pltpu.sync_copy(acc_sh.at[pl.ds(row0, NROWS)],
                    out_hbm.at[c, pl.ds(row0, NROWS)])


_sc_edge_scatter = functools.partial(
    pl.kernel,
    out_type=jax.ShapeDtypeStruct((NC, N, H), jnp.float32),
    mesh=_mesh,
    scratch_types=[
        pltpu.VMEM((KE,), jnp.int32),
        pltpu.VMEM((KE,), jnp.int32),
        pltpu.VMEM((KE, 16), jnp.float32),
        pltpu.VMEM((KE, H), jnp.float32),
        pltpu.VMEM_SHARED((N, H), jnp.float32),
        pltpu.SemaphoreType.DMA,
    ],
)(_sc_edge_scatter_body)


def _sc_degree_body(dst_hbm, ew16_hbm, zer_hbm, out_hbm,
                    dst_v, ew_v, acc_sh, sem):
    c = lax.axis_index("c")
    s = lax.axis_index("s")
    w = c * NS + s
    row0 = s * NROWS
    pltpu.sync_copy(zer_hbm.at[pl.ds(row0, NROWS)], acc_sh.at[pl.ds(row0, NROWS)])
    plsc.subcore_barrier()

    def chunk(g, carry):
        base = w * PER_TILE + g * KE
        pltpu.sync_copy(dst_hbm.at[pl.ds(base, KE)], dst_v)
        pltpu.sync_copy(ew16_hbm.at[pl.ds(base, KE)], ew_v)
        pltpu.sync_copy(ew_v, acc_sh.at[dst_v], add=True)
        return carry

    lax.fori_loop(0, CHUNKS, chunk, 0)
    plsc.subcore_barrier()
    pltpu.sync_copy(acc_sh.at[pl.ds(row0, NROWS)],
                    out_hbm.at[c, pl.ds(row0, NROWS)])


_sc_degree = functools.partial(
    pl.kernel,
    out_type=jax.ShapeDtypeStruct((NC, N, 16), jnp.float32),
    mesh=_mesh,
    scratch_types=[
        pltpu.VMEM((KE,), jnp.int32),
        pltpu.VMEM((KE, 16), jnp.float32),
        pltpu.VMEM_SHARED((N, 16), jnp.float32),
        pltpu.SemaphoreType.DMA,
    ],
)(_sc_degree_body)


RB = 1000
GRID = N // RB


def _sig(x):
    return 1.0 / (1.0 + jnp.exp(-x))


def _dot(a, b):
    return jnp.dot(a, b, preferred_element_type=jnp.float32)


def _gru_hs_body(x_ref, h_ref, dinv_ref, Win_ref, bin_ref, WihT_ref, bih_ref,
                 WhhT_ref, bhh_ref, Wc_ref, h1_ref, hs_ref):
    x = x_ref[...]
    h = h_ref[...]
    xp = _dot(x, Win_ref[...]) + bin_ref[...]
    gi = _dot(xp, WihT_ref[...]) + bih_ref[...]
    gh = _dot(h, WhhT_ref[...]) + bhh_ref[...]
    r = _sig(gi[:, 0:H] + gh[:, 0:H])
    z = _sig(gi[:, H:2 * H] + gh[:, H:2 * H])
    ng = jnp.tanh(gi[:, 2 * H:3 * H] + r * gh[:, 2 * H:3 * H])
    h1 = (1.0 - z) * ng + z * h
    h1_ref[...] = h1
    hs_ref[...] = _dot(h1, Wc_ref[...]) * dinv_ref[...]


def _combine_hs_body(S0_ref, S1_ref, hs_ref, hp_ref, dinv_ref, bc_ref,
                     Wc_ref, h2_ref, hs2_ref):
    dinv = dinv_ref[...]
    agg = (S0_ref[...] + S1_ref[...] + hs_ref[...]) * dinv + bc_ref[...]
    h2 = jnp.maximum(agg, 0.0) + hp_ref[...]
    h2_ref[...] = h2
    hs2_ref[...] = _dot(h2, Wc_ref[...]) * dinv


def _combine_out_body(S0_ref, S1_ref, hs_ref, hp_ref, dinv_ref, bc_ref,
                      Wlin_ref, blin_ref, h3_ref, out_ref):
    agg = (S0_ref[...] + S1_ref[...] + hs_ref[...]) * dinv_ref[...] + bc_ref[...]
    h3 = jnp.maximum(agg, 0.0) + hp_ref[...]
    h3_ref[...] = h3
    out_ref[...] = _dot(h3, Wlin_ref[...]) + blin_ref[...]


def _row_spec(width):
    return pl.BlockSpec((RB, width), lambda i: (i, 0))


def _full_spec(shape):
    return pl.BlockSpec(shape, lambda i: tuple(0 for _ in shape))


def _tc_call(body, args, row_widths, full_shapes, out_widths):
    in_specs = ([_row_spec(wd) for wd in row_widths]
                + [_full_spec(sh) for sh in full_shapes])
    out_specs = [_row_spec(wd) for wd in out_widths]
    out_shape = [jax.ShapeDtypeStruct((N, wd), jnp.float32) for wd in out_widths]
    return pl.pallas_call(
        body,
        grid=(GRID,),
        in_specs=in_specs,
        out_specs=out_specs,
        out_shape=out_shape,
        compiler_params=pltpu.CompilerParams(
            dimension_semantics=("arbitrary",)),
    )(*args)


def kernel(x_seq, edge_index, edge_weight, W_in, b_in, W_ih, W_hh, b_ih, b_hh,
           W_c1, b_c1, W_c2, b_c2, W_lin, b_lin):
    f32 = jnp.float32
    src = edge_index[0].astype(jnp.int32)
    dst = edge_index[1].astype(jnp.int32)
    pad = EPAD - E
    srcp = jnp.concatenate([src, jnp.zeros((pad,), jnp.int32)])
    dstp = jnp.concatenate([dst, jnp.zeros((pad,), jnp.int32)])
    ewp = jnp.concatenate([edge_weight.astype(f32), jnp.zeros((pad,), f32)])
    ew16 = jnp.broadcast_to(ewp[:, None], (EPAD, 16))
    zer128 = jnp.zeros((N, H), f32)
    zer16 = jnp.zeros((N, 16), f32)

    degp = _sc_degree(dstp, ew16, zer16)
    deg = degp[0, :, 0] + degp[1, :, 0] + 1.0
    dinv2 = lax.rsqrt(deg)[:, None]

    W_ihT = W_ih.T.astype(f32)
    W_hhT = W_hh.T.astype(f32)
    b_in2 = b_in.reshape(1, H).astype(f32)
    b_ih2 = b_ih.reshape(1, 3 * H).astype(f32)
    b_hh2 = b_hh.reshape(1, 3 * H).astype(f32)
    b_c1_2 = b_c1.reshape(1, H).astype(f32)
    b_c2_2 = b_c2.reshape(1, H).astype(f32)
    b_lin2 = b_lin.reshape(1, 1).astype(f32)

    h = jnp.zeros((N, H), f32)
    outs = []
    for t in range(T):
        h1, hs = _tc_call(
            _gru_hs_body,
            (x_seq[t], h, dinv2, W_in, b_in2, W_ihT, b_ih2, W_hhT, b_hh2, W_c1),
            (D, H, 1),
            ((D, H), (1, H), (H, 3 * H), (1, 3 * H), (H, 3 * H), (1, 3 * H), (H, H)),
            (H, H))
        S = _sc_edge_scatter(hs, srcp, dstp, ew16, zer128)
        h2, hs2 = _tc_call(
            _combine_hs_body,
            (S[0], S[1], hs, h1, dinv2, b_c1_2, W_c2),
            (H, H, H, H, 1),
            ((1, H), (H, H)),
            (H, H))
        S2 = _sc_edge_scatter(hs2, srcp, dstp, ew16, zer128)
        h, out_t = _tc_call(
            _combine_out_body,
            (S2[0], S2[1], hs2, h2, dinv2, b_c2_2, W_lin, b_lin2),
            (H, H, H, H, 1),
            ((1, H), (H, 1), (1, 1)),
            (H, 1))
        outs.append(out_t)
    return jnp.stack(outs, axis=0)


# speedup vs baseline: 4.1572x; 4.1572x over previous
import functools

import jax
import jax.numpy as jnp
from jax import lax
from jax.experimental import pallas as pl
from jax.experimental.pallas import tpu as pltpu
from jax.experimental.pallas import tpu_sc as plsc

N = 10000
E = 320000
T = 8
D = 128
H = 128

NC = 2
NS = 16
NW = NC * NS
KE = 128
CHUNKS = -(-E // (NW * KE))
EPAD = CHUNKS * NW * KE
PER_TILE = EPAD // NW
NPAD = 10240
NROWS = NPAD // NS


def _sc_edge_scatter_body(hs_hbm, src_hbm, dst_hbm, ew16_hbm, zer_hbm,
                          out_hbm, src_v, dst_v, ew_v, rows_v, acc_sh, sem):
    c = lax.axis_index("c")
    s = lax.axis_index("s")
    w = c * NS + s
    row0 = s * NROWS
    pltpu.sync_copy(zer_hbm.at[pl.ds(row0, NROWS)], acc_sh.at[pl.ds(row0, NROWS)])
    plsc.subcore_barrier()

    def chunk(g, carry):
        base = w * PER_TILE + g * KE
        pltpu.sync_copy(src_hbm.at[pl.ds(base, KE)], src_v)
        pltpu.sync_copy(dst_hbm.at[pl.ds(base, KE)], dst_v)
        pltpu.sync_copy(ew16_hbm.at[pl.ds(base, KE)], ew_v)
        pltpu.async_copy(hs_hbm.at[src_v], rows_v, sem).wait()

        def scale(jg, carry2):
            for u in range(8):
                j = jg * 8 + u
                wv = ew_v[j]
                for i in range(H // 16):
                    sl = pl.ds(i * 16, 16)
                    rows_v[j, sl] = rows_v[j, sl] * wv
            return carry2

        lax.fori_loop(0, KE // 8, scale, 0)
        pltpu.sync_copy(rows_v, acc_sh.at[dst_v], add=True)
        return carry

    lax.fori_loop(0, CHUNKS, chunk, 0)
    plsc.subcore_barrier()
    pltpu.sync_copy(acc_sh.at[pl.ds(row0, NROWS)],
                    out_hbm.at[c, pl.ds(row0, NROWS)])


@functools.lru_cache(maxsize=None)
def _sc_kernels():
    mesh = plsc.VectorSubcoreMesh(
        core_axis_name="c", subcore_axis_name="s",
        num_cores=NC, num_subcores=NS)
    edge_scatter = pl.kernel(
        _sc_edge_scatter_body,
        out_type=jax.ShapeDtypeStruct((NC, NPAD, H), jnp.float32),
        mesh=mesh,
        scratch_types=[
            pltpu.VMEM((KE,), jnp.int32),
            pltpu.VMEM((KE,), jnp.int32),
            pltpu.VMEM((KE, 16), jnp.float32),
            pltpu.VMEM((KE, H), jnp.float32),
            pltpu.VMEM_SHARED((NPAD, H), jnp.float32),
            pltpu.SemaphoreType.DMA,
        ],
    )
    return edge_scatter


def _sc_edge_scatter(*args):
    return _sc_kernels()(*args)


RB = 1000
GRID = N // RB


def _sig(x):
    return 1.0 / (1.0 + jnp.exp(-x))


def _dot(a, b):
    return jnp.dot(a, b, preferred_element_type=jnp.float32)


def _gru_hs_body(x_ref, h_ref, dinv_ref, Win_ref, bin_ref, WihT_ref, bih_ref,
                 WhhT_ref, bhh_ref, Wc_ref, h1_ref, hs_ref):
    x = x_ref[...]
    h = h_ref[...]
    xp = _dot(x, Win_ref[...]) + bin_ref[...]
    gi = _dot(xp, WihT_ref[...]) + bih_ref[...]
    gh = _dot(h, WhhT_ref[...]) + bhh_ref[...]
    r = _sig(gi[:, 0:H] + gh[:, 0:H])
    z = _sig(gi[:, H:2 * H] + gh[:, H:2 * H])
    ng = jnp.tanh(gi[:, 2 * H:3 * H] + r * gh[:, 2 * H:3 * H])
    h1 = (1.0 - z) * ng + z * h
    h1_ref[...] = h1
    hs_ref[...] = _dot(h1, Wc_ref[...]) * dinv_ref[...]


def _combine_hs_body(S0_ref, S1_ref, hs_ref, hp_ref, dinv_ref, bc_ref,
                     Wc_ref, h2_ref, hs2_ref):
    dinv = dinv_ref[...]
    agg = (S0_ref[...] + S1_ref[...] + hs_ref[...]) * dinv + bc_ref[...]
    h2 = jnp.maximum(agg, 0.0) + hp_ref[...]
    h2_ref[...] = h2
    hs2_ref[...] = _dot(h2, Wc_ref[...]) * dinv


def _combine_out_body(S0_ref, S1_ref, hs_ref, hp_ref, dinv_ref, bc_ref,
                      Wlin_ref, blin_ref, h3_ref, out_ref):
    agg = (S0_ref[...] + S1_ref[...] + hs_ref[...]) * dinv_ref[...] + bc_ref[...]
    h3 = jnp.maximum(agg, 0.0) + hp_ref[...]
    h3_ref[...] = h3
    out_ref[...] = _dot(h3, Wlin_ref[...]) + blin_ref[...]


def _row_spec(width):
    return pl.BlockSpec((RB, width), lambda i: (i, 0))


def _full_spec(shape):
    return pl.BlockSpec(shape, lambda i: tuple(0 for _ in shape))


def _tc_call(body, args, row_widths, full_shapes, out_widths):
    in_specs = ([_row_spec(wd) for wd in row_widths]
                + [_full_spec(sh) for sh in full_shapes])
    out_specs = [_row_spec(wd) for wd in out_widths]
    out_shape = [jax.ShapeDtypeStruct((N, wd), jnp.float32) for wd in out_widths]
    return pl.pallas_call(
        body,
        grid=(GRID,),
        in_specs=in_specs,
        out_specs=out_specs,
        out_shape=out_shape,
        compiler_params=pltpu.CompilerParams(
            dimension_semantics=("arbitrary",)),
    )(*args)


def kernel(x_seq, edge_index, edge_weight, W_in, b_in, W_ih, W_hh, b_ih, b_hh,
           W_c1, b_c1, W_c2, b_c2, W_lin, b_lin):
    f32 = jnp.float32
    src = edge_index[0].astype(jnp.int32)
    dst = edge_index[1].astype(jnp.int32)
    pad = EPAD - E
    srcp = jnp.concatenate([src, jnp.zeros((pad,), jnp.int32)])
    dstp = jnp.concatenate([dst, jnp.zeros((pad,), jnp.int32)])
    ewp = jnp.concatenate([edge_weight.astype(f32), jnp.zeros((pad,), f32)])
    ew16 = jnp.broadcast_to(ewp[:, None], (EPAD, 16))
    zer128 = jnp.zeros((NPAD, H), f32)
    ones128 = jnp.ones((N, H), f32)

    degp = _sc_edge_scatter(ones128, srcp, dstp, ew16, zer128)
    deg = degp[0, :N, 0] + degp[1, :N, 0] + 1.0
    dinv2 = lax.rsqrt(deg)[:, None]

    W_ihT = W_ih.T.astype(f32)
    W_hhT = W_hh.T.astype(f32)
    b_in2 = b_in.reshape(1, H).astype(f32)
    b_ih2 = b_ih.reshape(1, 3 * H).astype(f32)
    b_hh2 = b_hh.reshape(1, 3 * H).astype(f32)
    b_c1_2 = b_c1.reshape(1, H).astype(f32)
    b_c2_2 = b_c2.reshape(1, H).astype(f32)
    b_lin2 = b_lin.reshape(1, 1).astype(f32)

    h = jnp.zeros((N, H), f32)
    outs = []
    for t in range(T):
        h1, hs = _tc_call(
            _gru_hs_body,
            (x_seq[t], h, dinv2, W_in, b_in2, W_ihT, b_ih2, W_hhT, b_hh2, W_c1),
            (D, H, 1),
            ((D, H), (1, H), (H, 3 * H), (1, 3 * H), (H, 3 * H), (1, 3 * H), (H, H)),
            (H, H))
        Sp = _sc_edge_scatter(hs, srcp, dstp, ew16, zer128)
        S = Sp[:, :N]
        h2, hs2 = _tc_call(
            _combine_hs_body,
            (S[0], S[1], hs, h1, dinv2, b_c1_2, W_c2),
            (H, H, H, H, 1),
            ((1, H), (H, H)),
            (H, H))
        S2p = _sc_edge_scatter(hs2, srcp, dstp, ew16, zer128)
        S2 = S2p[:, :N]
        h, out_t = _tc_call(
            _combine_out_body,
            (S2[0], S2[1], hs2, h2, dinv2, b_c2_2, W_lin, b_lin2),
            (H, H, H, H, 1),
            ((1, H), (H, 1), (1, 1)),
            (H, 1))
        outs.append(out_t)
    return jnp.stack(outs, axis=0)
